# baseline (device time: 11795 ns/iter reference)
import jax
import jax.numpy as jnp
from jax import lax
from jax.experimental import pallas as pl
from jax.experimental.pallas import tpu as pltpu

N_DEV = 4


def kernel(x, W1, W2):
    m, k1 = x.shape
    _, h = W1.shape
    _, n = W2.shape

    def body(x_hbm, w1_hbm, w2_hbm, out_hbm,
             x_v, w1_v, w2_v, out_v, send_buf, comm_ref,
             in_sems, out_sem, send_sems, recv_sems):
        my = lax.axis_index("i")

        barrier = pltpu.get_barrier_semaphore()
        for k in range(1, N_DEV):
            peer = lax.rem(my + k, N_DEV)
            pl.semaphore_signal(
                barrier, inc=1,
                device_id=(peer,), device_id_type=pl.DeviceIdType.MESH,
            )

        cp_x = pltpu.make_async_copy(x_hbm, x_v, in_sems.at[0])
        cp_w1 = pltpu.make_async_copy(w1_hbm, w1_v, in_sems.at[1])
        cp_w2 = pltpu.make_async_copy(w2_hbm, w2_v, in_sems.at[2])
        cp_x.start()
        cp_w1.start()
        cp_w2.start()

        cp_x.wait()
        cp_w1.wait()
        xb = x_v[...].astype(jnp.bfloat16)
        w1 = w1_v[...].astype(jnp.bfloat16)
        hidden = lax.dot(xb, w1, preferred_element_type=jnp.float32)
        hidden = jnp.maximum(hidden, 0.0).astype(jnp.bfloat16)
        cp_w2.wait()
        w2 = w2_v[...].astype(jnp.bfloat16)
        partial = lax.dot(hidden, w2, preferred_element_type=jnp.float32)
        send_buf[...] = partial.astype(jnp.bfloat16)

        pl.semaphore_wait(barrier, N_DEV - 1)

        rdmas = {}
        for k in range(1, N_DEV):
            dst = lax.rem(my + k, N_DEV)
            j = N_DEV - 1 - k
            rdma = pltpu.make_async_remote_copy(
                src_ref=send_buf,
                dst_ref=comm_ref.at[j],
                send_sem=send_sems.at[k - 1],
                recv_sem=recv_sems.at[j],
                device_id=(dst,),
                device_id_type=pl.DeviceIdType.MESH,
            )
            rdma.start()
            rdmas[j] = rdma

        acc = partial
        for j in (2, 0, 1):
            rdmas[j].wait_recv()
            acc = acc + comm_ref[j].astype(jnp.float32)
        out_v[...] = acc.astype(jnp.bfloat16)

        cp_out = pltpu.make_async_copy(out_v, out_hbm, out_sem)
        cp_out.start()
        cp_out.wait()

        for rdma in rdmas.values():
            rdma.wait_send()

    return pl.pallas_call(
        body,
        out_shape=jax.ShapeDtypeStruct((m, n), jnp.bfloat16),
        in_specs=[
            pl.BlockSpec(memory_space=pl.ANY),
            pl.BlockSpec(memory_space=pl.ANY),
            pl.BlockSpec(memory_space=pl.ANY),
        ],
        out_specs=pl.BlockSpec(memory_space=pl.ANY),
        scratch_shapes=[
            pltpu.VMEM((m, k1), jnp.float32),
            pltpu.VMEM((k1, h), jnp.float32),
            pltpu.VMEM((h, n), jnp.float32),
            pltpu.VMEM((m, n), jnp.bfloat16),
            pltpu.VMEM((m, n), jnp.bfloat16),
            pltpu.VMEM((N_DEV - 1, m, n), jnp.bfloat16),
            pltpu.SemaphoreType.DMA((3,)),
            pltpu.SemaphoreType.DMA,
            pltpu.SemaphoreType.DMA((N_DEV - 1,)),
            pltpu.SemaphoreType.DMA((N_DEV - 1,)),
        ],
        compiler_params=pltpu.CompilerParams(collective_id=0),
    )(x, W1, W2)


# device time: 9166 ns/iter; 1.2868x vs baseline; 1.2868x over previous
import jax
import jax.numpy as jnp
from jax import lax
from jax.experimental import pallas as pl
from jax.experimental.pallas import tpu as pltpu

N_DEV = 4


def kernel(x, W1, W2):
    m, k1 = x.shape
    _, h = W1.shape
    _, n = W2.shape

    def body(x_hbm, w1_hbm, w2_hbm, out_hbm,
             x_v, w1_v, w2_v, out_v, send_buf, comm_ref,
             in_sems, out_sem, send_sems, recv_sems):
        my = lax.axis_index("i")

        barrier = pltpu.get_barrier_semaphore()
        for k in range(1, N_DEV):
            peer = lax.rem(my + k, N_DEV)
            pl.semaphore_signal(
                barrier, inc=1,
                device_id=(peer,), device_id_type=pl.DeviceIdType.MESH,
            )

        cp_x = pltpu.make_async_copy(x_hbm, x_v, in_sems.at[0])
        cp_w1 = pltpu.make_async_copy(w1_hbm, w1_v, in_sems.at[1])
        cp_w2 = pltpu.make_async_copy(w2_hbm, w2_v, in_sems.at[2])
        cp_x.start()
        cp_w1.start()
        cp_w2.start()

        cp_x.wait()
        cp_w1.wait()
        xb = x_v[...].astype(jnp.bfloat16)
        w1 = w1_v[...].astype(jnp.bfloat16)
        hidden = lax.dot(xb, w1, preferred_element_type=jnp.float32)
        hidden = jnp.maximum(hidden, 0.0).astype(jnp.bfloat16)
        cp_w2.wait()
        w2 = w2_v[...].astype(jnp.bfloat16)
        partial = lax.dot(hidden, w2, preferred_element_type=jnp.float32)
        send_buf[...] = partial.astype(jnp.bfloat16)

        pl.semaphore_wait(barrier, N_DEV - 1)

        rdmas = {}
        for k in range(1, N_DEV):
            dst = lax.rem(my + k, N_DEV)
            j = N_DEV - 1 - k
            rdma = pltpu.make_async_remote_copy(
                src_ref=send_buf,
                dst_ref=comm_ref.at[j],
                send_sem=send_sems.at[k - 1],
                recv_sem=recv_sems.at[j],
                device_id=(dst,),
                device_id_type=pl.DeviceIdType.MESH,
            )
            rdma.start()
            rdmas[j] = rdma

        acc = partial
        for j in (2, 0, 1):
            rdmas[j].wait_recv()
            acc = acc + comm_ref[j].astype(jnp.float32)
        out_v[...] = acc.astype(jnp.bfloat16)

        cp_out = pltpu.make_async_copy(out_v, out_hbm, out_sem)
        cp_out.start()
        cp_out.wait()

        for rdma in rdmas.values():
            rdma.wait_send()

    return pl.pallas_call(
        body,
        out_shape=jax.ShapeDtypeStruct((m, n), jnp.bfloat16),
        in_specs=[
            pl.BlockSpec(memory_space=pltpu.MemorySpace.HBM),
            pl.BlockSpec(memory_space=pltpu.MemorySpace.HBM),
            pl.BlockSpec(memory_space=pltpu.MemorySpace.HBM),
        ],
        out_specs=pl.BlockSpec(memory_space=pltpu.MemorySpace.HBM),
        scratch_shapes=[
            pltpu.VMEM((m, k1), jnp.float32),
            pltpu.VMEM((k1, h), jnp.float32),
            pltpu.VMEM((h, n), jnp.float32),
            pltpu.VMEM((m, n), jnp.bfloat16),
            pltpu.VMEM((m, n), jnp.bfloat16),
            pltpu.VMEM((N_DEV - 1, m, n), jnp.bfloat16),
            pltpu.SemaphoreType.DMA((3,)),
            pltpu.SemaphoreType.DMA,
            pltpu.SemaphoreType.DMA((N_DEV - 1,)),
            pltpu.SemaphoreType.DMA((N_DEV - 1,)),
        ],
        compiler_params=pltpu.CompilerParams(collective_id=0),
    )(
        pltpu.with_memory_space_constraint(x, pltpu.MemorySpace.HBM),
        pltpu.with_memory_space_constraint(W1, pltpu.MemorySpace.HBM),
        pltpu.with_memory_space_constraint(W2, pltpu.MemorySpace.HBM),
    )
